# baseline (speedup 1.0000x reference)
import jax
import jax.numpy as jnp
from jax import lax
from jax.experimental import pallas as pl
from jax.experimental.pallas import tpu as pltpu
from jax.experimental.pallas import tpu_sc as plsc

N = 10000
D = 128
N_EDGES = 320000
N_PAIRS = 65536

NC, NS, NW = 2, 16, 32
CHUNK = 128
NCH_E = -(-N_EDGES // (NW * CHUNK))
E_PAD = NW * NCH_E * CHUNK
NCHT = 2 * NCH_E
N0 = 48
N1 = NCHT - N0
SLABS_0 = ((0, 40), (40, N0 - 40))
SLABS_1 = ((0, 40), (40, 40), (80, N1 - 80))
NCH_P = (2 * N_PAIRS) // (NW * CHUNK)
N_ACC = 10240
ACC_DUMMY = N_ACC - 1
ROWS_SC = N_ACC // NS

_MESH = plsc.VectorSubcoreMesh(core_axis_name="c", subcore_axis_name="s")


def _wid():
    return lax.axis_index("c") * NS + lax.axis_index("s")


def _sc_degree_body(dst_hbm, ones_hbm, zeros_hbm, out_hbm, idx_v, ones_v, acc_sh, sem):
    cid = lax.axis_index("c")
    sid = lax.axis_index("s")
    row0 = sid * ROWS_SC
    pltpu.sync_copy(zeros_hbm.at[pl.ds(row0, ROWS_SC)], acc_sh.at[pl.ds(row0, ROWS_SC)])
    pltpu.sync_copy(ones_hbm, ones_v)
    pltpu.sync_copy(dst_hbm.at[_wid()], idx_v)
    plsc.subcore_barrier()

    def body(j, _):
        pltpu.sync_copy(ones_v, acc_sh.at[idx_v.at[j]], add=True)
        return _

    lax.fori_loop(0, NCH_E, body, None)
    plsc.subcore_barrier()
    pltpu.sync_copy(acc_sh.at[pl.ds(row0, ROWS_SC)], out_hbm.at[cid, pl.ds(row0, ROWS_SC)])


_sc_degree = pl.kernel(
    _sc_degree_body,
    out_type=jax.ShapeDtypeStruct((NC, N_ACC, D), jnp.float32),
    mesh=_MESH,
    scratch_types=[
        pltpu.VMEM((NCH_E, CHUNK), jnp.int32),
        pltpu.VMEM((CHUNK, D), jnp.float32),
        pltpu.VMEM_SHARED((N_ACC, D), jnp.float32),
        pltpu.SemaphoreType.DMA,
    ],
)


def _gather_pipeline(n, start_gather, wait_gather, consume):
    start_gather(0, 0)
    npairs = (n - 1) // 2

    def body(i, _):
        for b in range(2):
            j = 2 * i + b
            start_gather(j + 1, 1 - b)
            wait_gather(j, b)
            consume(j, b)
        return _

    lax.fori_loop(0, npairs, body, None)
    for jj in range(2 * npairs, n):
        b = jj % 2
        if jj + 1 < n:
            start_gather(jj + 1, 1 - b)
        wait_gather(jj, b)
        consume(jj, b)


def _sc_aggregate_body(g_hbm, src_hbm, dst_hbm, zeros_hbm, out_hbm,
                       src_v, dst_v, rows_v, acc_sh, sem0, sem1):
    cid = lax.axis_index("c")
    sid = lax.axis_index("s")
    row0 = sid * ROWS_SC
    pltpu.sync_copy(zeros_hbm.at[pl.ds(row0, ROWS_SC)], acc_sh.at[pl.ds(row0, ROWS_SC)])
    plsc.subcore_barrier()
    sems = (sem0, sem1)

    def run_slabs(col0, slabs):
        for base, n in slabs:
            pltpu.sync_copy(src_hbm.at[sid, pl.ds(col0 + base, n)],
                            src_v.at[pl.ds(0, n)])
            pltpu.sync_copy(dst_hbm.at[sid, pl.ds(col0 + base, n)],
                            dst_v.at[pl.ds(0, n)])

            def start_gather(j, b):
                pltpu.async_copy(g_hbm.at[src_v.at[j]], rows_v.at[b], sems[b])

            def wait_gather(j, b):
                pltpu.make_async_copy(g_hbm.at[src_v.at[j]], rows_v.at[b],
                                      sems[b]).wait()

            def consume(j, b):
                pltpu.sync_copy(rows_v.at[b], acc_sh.at[dst_v.at[j]], add=True)

            _gather_pipeline(n, start_gather, wait_gather, consume)

    @pl.when(cid == 0)
    def _():
        run_slabs(N0, SLABS_1)

    @pl.when(cid == 1)
    def _():
        run_slabs(0, SLABS_0)

    plsc.subcore_barrier()
    pltpu.sync_copy(acc_sh.at[pl.ds(row0, ROWS_SC)], out_hbm.at[cid, pl.ds(row0, ROWS_SC)])


_SLAB_MAX = max(n for _, n in SLABS_0 + SLABS_1)

_sc_aggregate = pl.kernel(
    _sc_aggregate_body,
    out_type=jax.ShapeDtypeStruct((NC, N_ACC, D), jnp.float32),
    mesh=_MESH,
    scratch_types=[
        pltpu.VMEM((_SLAB_MAX, CHUNK), jnp.int32),
        pltpu.VMEM((_SLAB_MAX, CHUNK), jnp.int32),
        pltpu.VMEM((2, CHUNK, D), jnp.float32),
        pltpu.VMEM_SHARED((N_ACC, D), jnp.float32),
        pltpu.SemaphoreType.DMA,
        pltpu.SemaphoreType.DMA,
    ],
)


NCH_PC = N_PAIRS // (NW * CHUNK)


def _sc_pair_gather_body(h_hbm, pairs_hbm, out_hbm, idx_v, rows_v, sem0, sem1):
    wid = _wid()
    base = wid * NCH_PC * CHUNK
    sems = (sem0, sem1)

    for col in range(2):
        pltpu.sync_copy(pairs_hbm.at[col, wid], idx_v)

        def start_gather(j, b):
            pltpu.async_copy(h_hbm.at[idx_v.at[j]], rows_v.at[b], sems[b])

        def wait_gather(j, b):
            pltpu.make_async_copy(h_hbm.at[idx_v.at[j]], rows_v.at[b], sems[b]).wait()

        def consume(j, b):
            pltpu.sync_copy(rows_v.at[b],
                            out_hbm.at[col, pl.ds(base + j * CHUNK, CHUNK)])

        _gather_pipeline(NCH_PC, start_gather, wait_gather, consume)


_sc_pair_gather = pl.kernel(
    _sc_pair_gather_body,
    out_type=jax.ShapeDtypeStruct((2, N_PAIRS, D), jnp.float32),
    mesh=_MESH,
    scratch_types=[
        pltpu.VMEM((NCH_PC, CHUNK), jnp.int32),
        pltpu.VMEM((2, CHUNK, D), jnp.float32),
        pltpu.SemaphoreType.DMA,
        pltpu.SemaphoreType.DMA,
    ],
)


_BR = 1000


def _tc1a_body(x_ref, w0_ref, h0_ref):
    h0_ref[...] = jnp.dot(x_ref[...], w0_ref[...], precision=lax.Precision.HIGHEST)


def _tc1a(x, W0):
    return pl.pallas_call(
        _tc1a_body,
        grid=(N // _BR,),
        in_specs=[
            pl.BlockSpec((_BR, D), lambda i: (i, 0)),
            pl.BlockSpec((D, D), lambda i: (0, 0)),
        ],
        out_specs=pl.BlockSpec((_BR, D), lambda i: (i, 0)),
        out_shape=jax.ShapeDtypeStruct((N, D), jnp.float32),
    )(x, W0)


def _tc1b_body(degp_ref, h0_ref, g0_ref, dinv_ref):
    deg = degp_ref[0, :, 0] + degp_ref[1, :, 0] + 1.0
    dinv = lax.rsqrt(deg)[:, None]
    g0_ref[...] = dinv * h0_ref[...]
    dinv_ref[...] = dinv


def _tc1b(degp, h0):
    return pl.pallas_call(
        _tc1b_body,
        grid=(N // _BR,),
        in_specs=[
            pl.BlockSpec((NC, _BR, D), lambda i: (0, i, 0)),
            pl.BlockSpec((_BR, D), lambda i: (i, 0)),
        ],
        out_specs=[
            pl.BlockSpec((_BR, D), lambda i: (i, 0)),
            pl.BlockSpec((_BR, 1), lambda i: (i, 0)),
        ],
        out_shape=[
            jax.ShapeDtypeStruct((N, D), jnp.float32),
            jax.ShapeDtypeStruct((N, 1), jnp.float32),
        ],
    )(degp, h0)


def _tc2_body(sp_ref, h_ref, dinv_ref, b_ref, w_ref, h1_ref, g1_ref):
    dinv = dinv_ref[...]
    t = dinv * (sp_ref[0] + sp_ref[1]) + dinv * dinv * h_ref[...] + b_ref[...]
    t = jnp.maximum(t, 0.0)
    h1 = jnp.dot(t, w_ref[...], precision=lax.Precision.HIGHEST)
    h1_ref[...] = h1
    g1_ref[...] = dinv * h1


def _tc2(Sp, h, dinv, b, W):
    return pl.pallas_call(
        _tc2_body,
        grid=(N // _BR,),
        in_specs=[
            pl.BlockSpec((NC, _BR, D), lambda i: (0, i, 0)),
            pl.BlockSpec((_BR, D), lambda i: (i, 0)),
            pl.BlockSpec((_BR, 1), lambda i: (i, 0)),
            pl.BlockSpec((1, D), lambda i: (0, 0)),
            pl.BlockSpec((D, D), lambda i: (0, 0)),
        ],
        out_specs=[
            pl.BlockSpec((_BR, D), lambda i: (i, 0)),
            pl.BlockSpec((_BR, D), lambda i: (i, 0)),
        ],
        out_shape=[
            jax.ShapeDtypeStruct((N, D), jnp.float32),
            jax.ShapeDtypeStruct((N, D), jnp.float32),
        ],
    )(Sp, h, dinv, b, W)


def _tc3_body(sp_ref, h_ref, dinv_ref, b_ref, h2_ref):
    dinv = dinv_ref[...]
    t = dinv * (sp_ref[0] + sp_ref[1]) + dinv * dinv * h_ref[...] + b_ref[...]
    h2_ref[...] = jnp.maximum(t, 0.0)


def _tc3(Sp, h, dinv, b):
    return pl.pallas_call(
        _tc3_body,
        grid=(N // _BR,),
        in_specs=[
            pl.BlockSpec((NC, _BR, D), lambda i: (0, i, 0)),
            pl.BlockSpec((_BR, D), lambda i: (i, 0)),
            pl.BlockSpec((_BR, 1), lambda i: (i, 0)),
            pl.BlockSpec((1, D), lambda i: (0, 0)),
        ],
        out_specs=pl.BlockSpec((_BR, D), lambda i: (i, 0)),
        out_shape=jax.ShapeDtypeStruct((N, D), jnp.float32),
    )(Sp, h, dinv, b)


_BP = 4096


def _tc4_body(p_ref, wa_ref, wb_ref, b1_ref, w2_ref, b2_ref, out_ref):
    z = (jnp.dot(p_ref[0], wa_ref[...])
         + jnp.dot(p_ref[1], wb_ref[...])
         + b1_ref[...])
    z = jnp.maximum(z, 0.0)
    out_ref[...] = jnp.sum(z * w2_ref[...], axis=1, keepdims=True) + b2_ref[...]


def _tc4(P, hWa, hWb, hb1, hW2row, hb2):
    return pl.pallas_call(
        _tc4_body,
        grid=(N_PAIRS // _BP,),
        in_specs=[
            pl.BlockSpec((2, _BP, D), lambda i: (0, i, 0)),
            pl.BlockSpec((D, D), lambda i: (0, 0)),
            pl.BlockSpec((D, D), lambda i: (0, 0)),
            pl.BlockSpec((1, D), lambda i: (0, 0)),
            pl.BlockSpec((1, D), lambda i: (0, 0)),
            pl.BlockSpec((1, 1), lambda i: (0, 0)),
        ],
        out_specs=pl.BlockSpec((_BP, 1), lambda i: (i, 0)),
        out_shape=jax.ShapeDtypeStruct((N_PAIRS, 1), jnp.float32),
    )(P, hWa, hWb, hb1, hW2row, hb2)


def kernel(x, edge_index, drug_pairs, W0, b0, W1, b1, hW1, hb1, hW2, hb2):
    n_pad = E_PAD - N_EDGES
    src_flat = jnp.concatenate(
        [edge_index[0].astype(jnp.int32), jnp.zeros((n_pad,), jnp.int32)])
    pad_dst = N + jnp.arange(n_pad, dtype=jnp.int32) % (N_ACC - N)
    dst_flat = jnp.concatenate([edge_index[1].astype(jnp.int32), pad_dst])
    dst_r = dst_flat.reshape(NW, NCH_E, CHUNK)
    src_t = src_flat.reshape(NS, NCHT, CHUNK)
    dst_t = dst_flat.reshape(NS, NCHT, CHUNK)
    pairs_r = drug_pairs.astype(jnp.int32).T.reshape(2, NW, NCH_PC, CHUNK)

    zeros_row = jnp.zeros((N_ACC, D), jnp.float32)
    ones_row = jnp.ones((CHUNK, D), jnp.float32)

    degp = _sc_degree(dst_r, ones_row, zeros_row)
    h0 = _tc1a(x, W0)
    g0, dinv = _tc1b(degp, h0)
    S0 = _sc_aggregate(g0, src_t, dst_t, zeros_row)
    h1, g1 = _tc2(S0, h0, dinv, b0.reshape(1, D), W1)
    S1 = _sc_aggregate(g1, src_t, dst_t, zeros_row)
    h2 = _tc3(S1, h1, dinv, b1.reshape(1, D))
    P = _sc_pair_gather(h2, pairs_r)
    out = _tc4(P, hW1[:D], hW1[D:], hb1.reshape(1, D),
               hW2.reshape(1, D), hb2.reshape(1, 1))
    return out


# speedup vs baseline: 1.1157x; 1.1157x over previous
import jax
import jax.numpy as jnp
from jax import lax
from jax.experimental import pallas as pl
from jax.experimental.pallas import tpu as pltpu
from jax.experimental.pallas import tpu_sc as plsc

N = 10000
D = 128
N_EDGES = 320000
N_PAIRS = 65536

NC, NS, NW = 2, 16, 32
CHUNK = 128
NCH_E = -(-N_EDGES // (NW * CHUNK))
E_PAD = NW * NCH_E * CHUNK
SLABS = ((0, 40), (40, NCH_E - 40))
NCH_P = (2 * N_PAIRS) // (NW * CHUNK)
N_ACC = 10240
ACC_DUMMY = N_ACC - 1
ROWS_SC = N_ACC // NS

_MESH = plsc.VectorSubcoreMesh(core_axis_name="c", subcore_axis_name="s")


def _wid():
    return lax.axis_index("c") * NS + lax.axis_index("s")


def _sc_degree_body(dst_hbm, ones_hbm, zeros_hbm, out_hbm, idx_v, ones_v, acc_sh, sem):
    cid = lax.axis_index("c")
    sid = lax.axis_index("s")
    row0 = sid * ROWS_SC
    pltpu.sync_copy(zeros_hbm.at[pl.ds(row0, ROWS_SC)], acc_sh.at[pl.ds(row0, ROWS_SC)])
    pltpu.sync_copy(ones_hbm, ones_v)
    pltpu.sync_copy(dst_hbm.at[_wid()], idx_v)
    plsc.subcore_barrier()

    def body(j, _):
        pltpu.sync_copy(ones_v, acc_sh.at[idx_v.at[j]], add=True)
        return _

    lax.fori_loop(0, NCH_E, body, None)
    plsc.subcore_barrier()
    pltpu.sync_copy(acc_sh.at[pl.ds(row0, ROWS_SC)], out_hbm.at[cid, pl.ds(row0, ROWS_SC)])


_sc_degree = pl.kernel(
    _sc_degree_body,
    out_type=jax.ShapeDtypeStruct((NC, N_ACC, D), jnp.float32),
    mesh=_MESH,
    scratch_types=[
        pltpu.VMEM((NCH_E, CHUNK), jnp.int32),
        pltpu.VMEM((CHUNK, D), jnp.float32),
        pltpu.VMEM_SHARED((N_ACC, D), jnp.float32),
        pltpu.SemaphoreType.DMA,
    ],
)


def _gather_pipeline(n, start_gather, wait_gather, consume):
    start_gather(0, 0)
    npairs = (n - 1) // 2

    def body(i, _):
        for b in range(2):
            j = 2 * i + b
            start_gather(j + 1, 1 - b)
            wait_gather(j, b)
            consume(j, b)
        return _

    lax.fori_loop(0, npairs, body, None)
    for jj in range(2 * npairs, n):
        b = jj % 2
        if jj + 1 < n:
            start_gather(jj + 1, 1 - b)
        wait_gather(jj, b)
        consume(jj, b)


def _sc_aggregate_body(g_hbm, src_hbm, dst_hbm, zeros_hbm, out_hbm,
                       src_v, dst_v, rows_v, acc_sh, sem0, sem1):
    cid = lax.axis_index("c")
    sid = lax.axis_index("s")
    wid = _wid()
    row0 = sid * ROWS_SC
    pltpu.sync_copy(zeros_hbm.at[pl.ds(row0, ROWS_SC)], acc_sh.at[pl.ds(row0, ROWS_SC)])
    plsc.subcore_barrier()
    sems = (sem0, sem1)

    for base, n in SLABS:
        pltpu.sync_copy(src_hbm.at[wid, pl.ds(base, n)], src_v.at[pl.ds(0, n)])
        pltpu.sync_copy(dst_hbm.at[wid, pl.ds(base, n)], dst_v.at[pl.ds(0, n)])

        def start_gather(j, b):
            pltpu.async_copy(g_hbm.at[src_v.at[j]], rows_v.at[b], sems[b])

        def wait_gather(j, b):
            pltpu.make_async_copy(g_hbm.at[src_v.at[j]], rows_v.at[b],
                                  sems[b]).wait()

        def consume(j, b):
            pltpu.sync_copy(rows_v.at[b], acc_sh.at[dst_v.at[j]], add=True)

        _gather_pipeline(n, start_gather, wait_gather, consume)

    plsc.subcore_barrier()
    pltpu.sync_copy(acc_sh.at[pl.ds(row0, ROWS_SC)], out_hbm.at[cid, pl.ds(row0, ROWS_SC)])


_SLAB_MAX = max(n for _, n in SLABS)

_sc_aggregate = pl.kernel(
    _sc_aggregate_body,
    out_type=jax.ShapeDtypeStruct((NC, N_ACC, D), jnp.float32),
    mesh=_MESH,
    scratch_types=[
        pltpu.VMEM((_SLAB_MAX, CHUNK), jnp.int32),
        pltpu.VMEM((_SLAB_MAX, CHUNK), jnp.int32),
        pltpu.VMEM((2, CHUNK, D), jnp.float32),
        pltpu.VMEM_SHARED((N_ACC, D), jnp.float32),
        pltpu.SemaphoreType.DMA,
        pltpu.SemaphoreType.DMA,
    ],
)


NCH_PC = N_PAIRS // (NW * CHUNK)


def _sc_pair_gather_body(h_hbm, pairs_hbm, out_hbm, idx_v, rows_v, sem0, sem1):
    wid = _wid()
    base = wid * NCH_PC * CHUNK
    sems = (sem0, sem1)

    for col in range(2):
        pltpu.sync_copy(pairs_hbm.at[col, wid], idx_v)

        def start_gather(j, b):
            pltpu.async_copy(h_hbm.at[idx_v.at[j]], rows_v.at[b], sems[b])

        def wait_gather(j, b):
            pltpu.make_async_copy(h_hbm.at[idx_v.at[j]], rows_v.at[b], sems[b]).wait()

        def consume(j, b):
            pltpu.sync_copy(rows_v.at[b],
                            out_hbm.at[col, pl.ds(base + j * CHUNK, CHUNK)])

        _gather_pipeline(NCH_PC, start_gather, wait_gather, consume)


_sc_pair_gather = pl.kernel(
    _sc_pair_gather_body,
    out_type=jax.ShapeDtypeStruct((2, N_PAIRS, D), jnp.float32),
    mesh=_MESH,
    scratch_types=[
        pltpu.VMEM((NCH_PC, CHUNK), jnp.int32),
        pltpu.VMEM((2, CHUNK, D), jnp.float32),
        pltpu.SemaphoreType.DMA,
        pltpu.SemaphoreType.DMA,
    ],
)


_BR = 1000


def _tc1a_body(x_ref, w0_ref, h0_ref):
    h0_ref[...] = jnp.dot(x_ref[...], w0_ref[...], precision=lax.Precision.HIGHEST)


def _tc1a(x, W0):
    return pl.pallas_call(
        _tc1a_body,
        grid=(N // _BR,),
        in_specs=[
            pl.BlockSpec((_BR, D), lambda i: (i, 0)),
            pl.BlockSpec((D, D), lambda i: (0, 0)),
        ],
        out_specs=pl.BlockSpec((_BR, D), lambda i: (i, 0)),
        out_shape=jax.ShapeDtypeStruct((N, D), jnp.float32),
    )(x, W0)


def _tc1b_body(degp_ref, h0_ref, g0_ref, dinv_ref):
    deg = degp_ref[0, :, 0] + degp_ref[1, :, 0] + 1.0
    dinv = lax.rsqrt(deg)[:, None]
    g0_ref[...] = dinv * h0_ref[...]
    dinv_ref[...] = dinv


def _tc1b(degp, h0):
    return pl.pallas_call(
        _tc1b_body,
        grid=(N // _BR,),
        in_specs=[
            pl.BlockSpec((NC, _BR, D), lambda i: (0, i, 0)),
            pl.BlockSpec((_BR, D), lambda i: (i, 0)),
        ],
        out_specs=[
            pl.BlockSpec((_BR, D), lambda i: (i, 0)),
            pl.BlockSpec((_BR, 1), lambda i: (i, 0)),
        ],
        out_shape=[
            jax.ShapeDtypeStruct((N, D), jnp.float32),
            jax.ShapeDtypeStruct((N, 1), jnp.float32),
        ],
    )(degp, h0)


def _tc2_body(sp_ref, h_ref, dinv_ref, b_ref, w_ref, h1_ref, g1_ref):
    dinv = dinv_ref[...]
    t = dinv * (sp_ref[0] + sp_ref[1]) + dinv * dinv * h_ref[...] + b_ref[...]
    t = jnp.maximum(t, 0.0)
    h1 = jnp.dot(t, w_ref[...], precision=lax.Precision.HIGHEST)
    h1_ref[...] = h1
    g1_ref[...] = dinv * h1


def _tc2(Sp, h, dinv, b, W):
    return pl.pallas_call(
        _tc2_body,
        grid=(N // _BR,),
        in_specs=[
            pl.BlockSpec((NC, _BR, D), lambda i: (0, i, 0)),
            pl.BlockSpec((_BR, D), lambda i: (i, 0)),
            pl.BlockSpec((_BR, 1), lambda i: (i, 0)),
            pl.BlockSpec((1, D), lambda i: (0, 0)),
            pl.BlockSpec((D, D), lambda i: (0, 0)),
        ],
        out_specs=[
            pl.BlockSpec((_BR, D), lambda i: (i, 0)),
            pl.BlockSpec((_BR, D), lambda i: (i, 0)),
        ],
        out_shape=[
            jax.ShapeDtypeStruct((N, D), jnp.float32),
            jax.ShapeDtypeStruct((N, D), jnp.float32),
        ],
    )(Sp, h, dinv, b, W)


def _tc3_body(sp_ref, h_ref, dinv_ref, b_ref, h2_ref):
    dinv = dinv_ref[...]
    t = dinv * (sp_ref[0] + sp_ref[1]) + dinv * dinv * h_ref[...] + b_ref[...]
    h2_ref[...] = jnp.maximum(t, 0.0)


def _tc3(Sp, h, dinv, b):
    return pl.pallas_call(
        _tc3_body,
        grid=(N // _BR,),
        in_specs=[
            pl.BlockSpec((NC, _BR, D), lambda i: (0, i, 0)),
            pl.BlockSpec((_BR, D), lambda i: (i, 0)),
            pl.BlockSpec((_BR, 1), lambda i: (i, 0)),
            pl.BlockSpec((1, D), lambda i: (0, 0)),
        ],
        out_specs=pl.BlockSpec((_BR, D), lambda i: (i, 0)),
        out_shape=jax.ShapeDtypeStruct((N, D), jnp.float32),
    )(Sp, h, dinv, b)


_BP = 4096


def _tc4_body(p_ref, wa_ref, wb_ref, b1_ref, w2_ref, b2_ref, out_ref):
    z = (jnp.dot(p_ref[0], wa_ref[...])
         + jnp.dot(p_ref[1], wb_ref[...])
         + b1_ref[...])
    z = jnp.maximum(z, 0.0)
    out_ref[...] = jnp.sum(z * w2_ref[...], axis=1, keepdims=True) + b2_ref[...]


def _tc4(P, hWa, hWb, hb1, hW2row, hb2):
    return pl.pallas_call(
        _tc4_body,
        grid=(N_PAIRS // _BP,),
        in_specs=[
            pl.BlockSpec((2, _BP, D), lambda i: (0, i, 0)),
            pl.BlockSpec((D, D), lambda i: (0, 0)),
            pl.BlockSpec((D, D), lambda i: (0, 0)),
            pl.BlockSpec((1, D), lambda i: (0, 0)),
            pl.BlockSpec((1, D), lambda i: (0, 0)),
            pl.BlockSpec((1, 1), lambda i: (0, 0)),
        ],
        out_specs=pl.BlockSpec((_BP, 1), lambda i: (i, 0)),
        out_shape=jax.ShapeDtypeStruct((N_PAIRS, 1), jnp.float32),
    )(P, hWa, hWb, hb1, hW2row, hb2)


def kernel(x, edge_index, drug_pairs, W0, b0, W1, b1, hW1, hb1, hW2, hb2):
    n_pad = E_PAD - N_EDGES
    src_flat = jnp.concatenate(
        [edge_index[0].astype(jnp.int32), jnp.zeros((n_pad,), jnp.int32)])
    pad_dst = N + jnp.arange(n_pad, dtype=jnp.int32) % (N_ACC - N)
    dst_flat = jnp.concatenate([edge_index[1].astype(jnp.int32), pad_dst])
    dst_r = dst_flat.reshape(NW, NCH_E, CHUNK)
    src_t = src_flat.reshape(NW, NCH_E, CHUNK)
    dst_t = dst_r
    pairs_r = drug_pairs.astype(jnp.int32).T.reshape(2, NW, NCH_PC, CHUNK)

    zeros_row = jnp.zeros((N_ACC, D), jnp.float32)
    ones_row = jnp.ones((CHUNK, D), jnp.float32)

    degp = _sc_degree(dst_r, ones_row, zeros_row)
    h0 = _tc1a(x, W0)
    g0, dinv = _tc1b(degp, h0)
    S0 = _sc_aggregate(g0, src_t, dst_t, zeros_row)
    h1, g1 = _tc2(S0, h0, dinv, b0.reshape(1, D), W1)
    S1 = _sc_aggregate(g1, src_t, dst_t, zeros_row)
    h2 = _tc3(S1, h1, dinv, b1.reshape(1, D))
    P = _sc_pair_gather(h2, pairs_r)
    out = _tc4(P, hW1[:D], hW1[D:], hb1.reshape(1, D),
               hW2.reshape(1, D), hb2.reshape(1, 1))
    return out


# speedup vs baseline: 1.1267x; 1.0099x over previous
import jax
import jax.numpy as jnp
from jax import lax
from jax.experimental import pallas as pl
from jax.experimental.pallas import tpu as pltpu
from jax.experimental.pallas import tpu_sc as plsc

N = 10000
D = 128
N_EDGES = 320000
N_PAIRS = 65536

NC, NS, NW = 2, 16, 32
CHUNK = 128
NCH_E = -(-N_EDGES // (NW * CHUNK))
E_PAD = NW * NCH_E * CHUNK
SLABS = ((0, 40), (40, NCH_E - 40))
NCH_P = (2 * N_PAIRS) // (NW * CHUNK)
N_ACC = 10240
ACC_DUMMY = N_ACC - 1
ROWS_SC = N_ACC // NS

_MESH = plsc.VectorSubcoreMesh(core_axis_name="c", subcore_axis_name="s")


def _wid():
    return lax.axis_index("c") * NS + lax.axis_index("s")


def _sc_degree_body(dst_hbm, ones_hbm, zeros_hbm, out_hbm, idx_v, ones_v, acc_sh, sem):
    cid = lax.axis_index("c")
    sid = lax.axis_index("s")
    row0 = sid * ROWS_SC
    pltpu.sync_copy(zeros_hbm.at[pl.ds(row0, ROWS_SC)], acc_sh.at[pl.ds(row0, ROWS_SC)])
    pltpu.sync_copy(ones_hbm, ones_v)
    pltpu.sync_copy(dst_hbm.at[_wid()], idx_v)
    plsc.subcore_barrier()

    def body(j, _):
        pltpu.sync_copy(ones_v, acc_sh.at[idx_v.at[j]], add=True)
        return _

    lax.fori_loop(0, NCH_E, body, None)
    plsc.subcore_barrier()
    pltpu.sync_copy(acc_sh.at[pl.ds(row0, ROWS_SC)], out_hbm.at[cid, pl.ds(row0, ROWS_SC)])


_sc_degree = pl.kernel(
    _sc_degree_body,
    out_type=jax.ShapeDtypeStruct((NC, N_ACC, D), jnp.float32),
    mesh=_MESH,
    scratch_types=[
        pltpu.VMEM((NCH_E, CHUNK), jnp.int32),
        pltpu.VMEM((CHUNK, D), jnp.float32),
        pltpu.VMEM_SHARED((N_ACC, D), jnp.float32),
        pltpu.SemaphoreType.DMA,
    ],
)


def _gather_pipeline(n, start_gather, wait_gather, consume):
    start_gather(0, 0)
    npairs = (n - 1) // 2

    def body(i, _):
        for b in range(2):
            j = 2 * i + b
            start_gather(j + 1, 1 - b)
            wait_gather(j, b)
            consume(j, b)
        return _

    lax.fori_loop(0, npairs, body, None)
    for jj in range(2 * npairs, n):
        b = jj % 2
        if jj + 1 < n:
            start_gather(jj + 1, 1 - b)
        wait_gather(jj, b)
        consume(jj, b)


_NBUF = 4
_DEPTH = 3


def _sc_aggregate_body(g_hbm, src_hbm, dst_hbm, zeros_hbm, out_hbm,
                       src_v, dst_v, rows_v, acc_sh, sem0, sem1, sem2, sem3):
    cid = lax.axis_index("c")
    sid = lax.axis_index("s")
    wid = _wid()
    row0 = sid * ROWS_SC
    pltpu.sync_copy(zeros_hbm.at[pl.ds(row0, ROWS_SC)], acc_sh.at[pl.ds(row0, ROWS_SC)])
    plsc.subcore_barrier()
    sems = (sem0, sem1, sem2, sem3)

    for base, n in SLABS:
        pltpu.sync_copy(src_hbm.at[wid, pl.ds(base, n)], src_v.at[pl.ds(0, n)])
        pltpu.sync_copy(dst_hbm.at[wid, pl.ds(base, n)], dst_v.at[pl.ds(0, n)])

        def start_gather(u, b):
            j, h = divmod(u, 2)
            pltpu.async_copy(g_hbm.at[src_v.at[j, pl.ds(h * 64, 64)]],
                             rows_v.at[b], sems[b])

        def wait_gather(u, b):
            j, h = divmod(u, 2)
            pltpu.make_async_copy(g_hbm.at[src_v.at[j, pl.ds(h * 64, 64)]],
                                  rows_v.at[b], sems[b]).wait()

        def consume(u, b):
            j, h = divmod(u, 2)
            pltpu.sync_copy(rows_v.at[b],
                            acc_sh.at[dst_v.at[j, pl.ds(h * 64, 64)]], add=True)

        nu = 2 * n
        for u in range(min(_DEPTH, nu)):
            start_gather(u, u % _NBUF)
        for u in range(nu):
            if u + _DEPTH < nu:
                start_gather(u + _DEPTH, (u + _DEPTH) % _NBUF)
            wait_gather(u, u % _NBUF)
            consume(u, u % _NBUF)

    plsc.subcore_barrier()
    pltpu.sync_copy(acc_sh.at[pl.ds(row0, ROWS_SC)], out_hbm.at[cid, pl.ds(row0, ROWS_SC)])


_SLAB_MAX = max(n for _, n in SLABS)

_sc_aggregate = pl.kernel(
    _sc_aggregate_body,
    out_type=jax.ShapeDtypeStruct((NC, N_ACC, D), jnp.float32),
    mesh=_MESH,
    scratch_types=[
        pltpu.VMEM((_SLAB_MAX, CHUNK), jnp.int32),
        pltpu.VMEM((_SLAB_MAX, CHUNK), jnp.int32),
        pltpu.VMEM((_NBUF, CHUNK // 2, D), jnp.float32),
        pltpu.VMEM_SHARED((N_ACC, D), jnp.float32),
        pltpu.SemaphoreType.DMA,
        pltpu.SemaphoreType.DMA,
        pltpu.SemaphoreType.DMA,
        pltpu.SemaphoreType.DMA,
    ],
)


NCH_PC = N_PAIRS // (NW * CHUNK)


def _sc_pair_gather_body(h_hbm, pairs_hbm, out_hbm, idx_v, rows_v, sem0, sem1):
    wid = _wid()
    base = wid * NCH_PC * CHUNK
    sems = (sem0, sem1)

    for col in range(2):
        pltpu.sync_copy(pairs_hbm.at[col, wid], idx_v)

        def start_gather(j, b):
            pltpu.async_copy(h_hbm.at[idx_v.at[j]], rows_v.at[b], sems[b])

        def wait_gather(j, b):
            pltpu.make_async_copy(h_hbm.at[idx_v.at[j]], rows_v.at[b], sems[b]).wait()

        def consume(j, b):
            pltpu.sync_copy(rows_v.at[b],
                            out_hbm.at[col, pl.ds(base + j * CHUNK, CHUNK)])

        _gather_pipeline(NCH_PC, start_gather, wait_gather, consume)


_sc_pair_gather = pl.kernel(
    _sc_pair_gather_body,
    out_type=jax.ShapeDtypeStruct((2, N_PAIRS, D), jnp.float32),
    mesh=_MESH,
    scratch_types=[
        pltpu.VMEM((NCH_PC, CHUNK), jnp.int32),
        pltpu.VMEM((2, CHUNK, D), jnp.float32),
        pltpu.SemaphoreType.DMA,
        pltpu.SemaphoreType.DMA,
    ],
)


_BR = 1000


def _tc1a_body(x_ref, w0_ref, h0_ref):
    h0_ref[...] = jnp.dot(x_ref[...], w0_ref[...], precision=lax.Precision.HIGHEST)


def _tc1a(x, W0):
    return pl.pallas_call(
        _tc1a_body,
        grid=(N // _BR,),
        in_specs=[
            pl.BlockSpec((_BR, D), lambda i: (i, 0)),
            pl.BlockSpec((D, D), lambda i: (0, 0)),
        ],
        out_specs=pl.BlockSpec((_BR, D), lambda i: (i, 0)),
        out_shape=jax.ShapeDtypeStruct((N, D), jnp.float32),
    )(x, W0)


def _tc1b_body(degp_ref, h0_ref, g0_ref, dinv_ref):
    deg = degp_ref[0, :, 0] + degp_ref[1, :, 0] + 1.0
    dinv = lax.rsqrt(deg)[:, None]
    g0_ref[...] = dinv * h0_ref[...]
    dinv_ref[...] = dinv


def _tc1b(degp, h0):
    return pl.pallas_call(
        _tc1b_body,
        grid=(N // _BR,),
        in_specs=[
            pl.BlockSpec((NC, _BR, D), lambda i: (0, i, 0)),
            pl.BlockSpec((_BR, D), lambda i: (i, 0)),
        ],
        out_specs=[
            pl.BlockSpec((_BR, D), lambda i: (i, 0)),
            pl.BlockSpec((_BR, 1), lambda i: (i, 0)),
        ],
        out_shape=[
            jax.ShapeDtypeStruct((N, D), jnp.float32),
            jax.ShapeDtypeStruct((N, 1), jnp.float32),
        ],
    )(degp, h0)


def _tc2_body(sp_ref, h_ref, dinv_ref, b_ref, w_ref, h1_ref, g1_ref):
    dinv = dinv_ref[...]
    t = dinv * (sp_ref[0] + sp_ref[1]) + dinv * dinv * h_ref[...] + b_ref[...]
    t = jnp.maximum(t, 0.0)
    h1 = jnp.dot(t, w_ref[...], precision=lax.Precision.HIGHEST)
    h1_ref[...] = h1
    g1_ref[...] = dinv * h1


def _tc2(Sp, h, dinv, b, W):
    return pl.pallas_call(
        _tc2_body,
        grid=(N // _BR,),
        in_specs=[
            pl.BlockSpec((NC, _BR, D), lambda i: (0, i, 0)),
            pl.BlockSpec((_BR, D), lambda i: (i, 0)),
            pl.BlockSpec((_BR, 1), lambda i: (i, 0)),
            pl.BlockSpec((1, D), lambda i: (0, 0)),
            pl.BlockSpec((D, D), lambda i: (0, 0)),
        ],
        out_specs=[
            pl.BlockSpec((_BR, D), lambda i: (i, 0)),
            pl.BlockSpec((_BR, D), lambda i: (i, 0)),
        ],
        out_shape=[
            jax.ShapeDtypeStruct((N, D), jnp.float32),
            jax.ShapeDtypeStruct((N, D), jnp.float32),
        ],
    )(Sp, h, dinv, b, W)


def _tc3_body(sp_ref, h_ref, dinv_ref, b_ref, h2_ref):
    dinv = dinv_ref[...]
    t = dinv * (sp_ref[0] + sp_ref[1]) + dinv * dinv * h_ref[...] + b_ref[...]
    h2_ref[...] = jnp.maximum(t, 0.0)


def _tc3(Sp, h, dinv, b):
    return pl.pallas_call(
        _tc3_body,
        grid=(N // _BR,),
        in_specs=[
            pl.BlockSpec((NC, _BR, D), lambda i: (0, i, 0)),
            pl.BlockSpec((_BR, D), lambda i: (i, 0)),
            pl.BlockSpec((_BR, 1), lambda i: (i, 0)),
            pl.BlockSpec((1, D), lambda i: (0, 0)),
        ],
        out_specs=pl.BlockSpec((_BR, D), lambda i: (i, 0)),
        out_shape=jax.ShapeDtypeStruct((N, D), jnp.float32),
    )(Sp, h, dinv, b)


_BP = 4096


def _tc4_body(p_ref, wa_ref, wb_ref, b1_ref, w2_ref, b2_ref, out_ref):
    z = (jnp.dot(p_ref[0], wa_ref[...])
         + jnp.dot(p_ref[1], wb_ref[...])
         + b1_ref[...])
    z = jnp.maximum(z, 0.0)
    out_ref[...] = jnp.sum(z * w2_ref[...], axis=1, keepdims=True) + b2_ref[...]


def _tc4(P, hWa, hWb, hb1, hW2row, hb2):
    return pl.pallas_call(
        _tc4_body,
        grid=(N_PAIRS // _BP,),
        in_specs=[
            pl.BlockSpec((2, _BP, D), lambda i: (0, i, 0)),
            pl.BlockSpec((D, D), lambda i: (0, 0)),
            pl.BlockSpec((D, D), lambda i: (0, 0)),
            pl.BlockSpec((1, D), lambda i: (0, 0)),
            pl.BlockSpec((1, D), lambda i: (0, 0)),
            pl.BlockSpec((1, 1), lambda i: (0, 0)),
        ],
        out_specs=pl.BlockSpec((_BP, 1), lambda i: (i, 0)),
        out_shape=jax.ShapeDtypeStruct((N_PAIRS, 1), jnp.float32),
    )(P, hWa, hWb, hb1, hW2row, hb2)


def kernel(x, edge_index, drug_pairs, W0, b0, W1, b1, hW1, hb1, hW2, hb2):
    n_pad = E_PAD - N_EDGES
    src_flat = jnp.concatenate(
        [edge_index[0].astype(jnp.int32), jnp.zeros((n_pad,), jnp.int32)])
    pad_dst = N + jnp.arange(n_pad, dtype=jnp.int32) % (N_ACC - N)
    dst_flat = jnp.concatenate([edge_index[1].astype(jnp.int32), pad_dst])
    dst_r = dst_flat.reshape(NW, NCH_E, CHUNK)
    src_t = src_flat.reshape(NW, NCH_E, CHUNK)
    dst_t = dst_r
    pairs_r = drug_pairs.astype(jnp.int32).T.reshape(2, NW, NCH_PC, CHUNK)

    zeros_row = jnp.zeros((N_ACC, D), jnp.float32)
    ones_row = jnp.ones((CHUNK, D), jnp.float32)

    degp = _sc_degree(dst_r, ones_row, zeros_row)
    h0 = _tc1a(x, W0)
    g0, dinv = _tc1b(degp, h0)
    S0 = _sc_aggregate(g0, src_t, dst_t, zeros_row)
    h1, g1 = _tc2(S0, h0, dinv, b0.reshape(1, D), W1)
    S1 = _sc_aggregate(g1, src_t, dst_t, zeros_row)
    h2 = _tc3(S1, h1, dinv, b1.reshape(1, D))
    P = _sc_pair_gather(h2, pairs_r)
    out = _tc4(P, hW1[:D], hW1[D:], hb1.reshape(1, D),
               hW2.reshape(1, D), hb2.reshape(1, 1))
    return out
